# Initial kernel scaffold; baseline (speedup 1.0000x reference)
import jax
import jax.numpy as jnp
from jax.experimental import pallas as pl


def kernel(boxes, scores):
    raise NotImplementedError("write your pallas kernel here")


# speedup vs baseline: 1.1436x; 1.1436x over previous
import functools

import jax
import jax.numpy as jnp
from jax import lax
from jax.experimental import pallas as pl

_N = 20000
_K = 5000
_T = 512
_KP = 5120
_NSTRIP = _KP // _T
_THR = 0.5


def _nms_body(rows_ref, cols_ref, out_ref):
    i = pl.program_id(0)
    r = rows_ref[...]
    cxr = r[:, 0:1]
    cyr = r[:, 1:2]
    wr = r[:, 3:4]
    lr = r[:, 4:5]
    x1r = cxr - wr * 0.5
    x2r = cxr + wr * 0.5
    y1r = cyr - lr * 0.5
    y2r = cyr + lr * 0.5
    arear = wr * lr
    row_ids = i * _T + lax.broadcasted_iota(jnp.int32, (_T, 1), 0)

    def body(j, m):
        c = cols_ref[:, pl.ds(j * _T, _T)]
        cxc = c[0:1, :]
        cyc = c[1:2, :]
        wc = c[3:4, :]
        lc = c[4:5, :]
        x1c = cxc - wc * 0.5
        x2c = cxc + wc * 0.5
        y1c = cyc - lc * 0.5
        y2c = cyc + lc * 0.5
        areac = wc * lc
        ix = jnp.maximum(
            jnp.minimum(x2r, x2c) - jnp.maximum(x1r, x1c), 0.0)
        iy = jnp.maximum(
            jnp.minimum(y2r, y2c) - jnp.maximum(y1r, y1c), 0.0)
        inter = ix * iy
        s = inter - _THR * (arear + areac - inter)
        col_ids = j * _T + lax.broadcasted_iota(jnp.int32, (1, _T), 1)
        s = jnp.where(col_ids < row_ids, s, -1.0)
        return jnp.maximum(m, jnp.max(s, axis=1, keepdims=True))

    m = lax.fori_loop(0, i + 1, body, jnp.full((_T, 1), -1.0, jnp.float32))
    keep = (m <= 0.0).astype(jnp.float32)
    out_ref[...] = r * keep


_nms_call = pl.pallas_call(
    _nms_body,
    grid=(_NSTRIP,),
    in_specs=[
        pl.BlockSpec((_T, 8), lambda i: (i, 0)),
        pl.BlockSpec((8, _KP), lambda i: (0, 0)),
    ],
    out_specs=pl.BlockSpec((_T, 8), lambda i: (i, 0)),
    out_shape=jax.ShapeDtypeStruct((_KP, 8), jnp.float32),
)


@jax.jit
def kernel(boxes, scores):
    top_scores, top_idx = lax.top_k(scores, _K)
    top_boxes = jnp.take(boxes, top_idx, axis=0)
    rows = jnp.zeros((_KP, 8), jnp.float32)
    rows = rows.at[:_K, :6].set(top_boxes)
    rows = rows.at[:_K, 6].set(top_scores)
    cols = rows.T
    out = _nms_call(rows, cols)
    return out[:_K, :7]


# speedup vs baseline: 1.2298x; 1.0753x over previous
import functools

import jax
import jax.numpy as jnp
from jax import lax
from jax.experimental import pallas as pl

_N = 20000
_K = 5000
_T = 512
_KP = 5120
_NSTRIP = _KP // _T
_THR = 0.5


def _nms_body(rows_ref, scores_ref, cols_ref, out_ref):
    i = pl.program_id(0)
    r = rows_ref[...]
    cxr = r[:, 0:1]
    cyr = r[:, 1:2]
    wr = r[:, 3:4]
    lr = r[:, 4:5]
    x1r = cxr - wr * 0.5
    x2r = cxr + wr * 0.5
    y1r = cyr - lr * 0.5
    y2r = cyr + lr * 0.5
    ar2 = wr * lr * _THR

    def tile_s(j):
        c = cols_ref[:, pl.ds(j * _T, _T)]
        cxc = c[0:1, :]
        cyc = c[1:2, :]
        wc = c[3:4, :]
        lc = c[4:5, :]
        x1c = cxc - wc * 0.5
        x2c = cxc + wc * 0.5
        y1c = cyc - lc * 0.5
        y2c = cyc + lc * 0.5
        ac2 = wc * lc * _THR
        ix = jnp.maximum(
            jnp.minimum(x2r, x2c) - jnp.maximum(x1r, x1c), 0.0)
        iy = jnp.maximum(
            jnp.minimum(y2r, y2c) - jnp.maximum(y1r, y1c), 0.0)
        inter = ix * iy
        return (1.0 + _THR) * inter - (ar2 + ac2)

    def body(j, m):
        return jnp.maximum(m, tile_s(j))

    m = lax.fori_loop(0, i, body, jnp.full((_T, _T), -1.0, jnp.float32))
    sd = tile_s(i)
    row_ids = lax.broadcasted_iota(jnp.int32, (_T, 1), 0)
    col_ids = lax.broadcasted_iota(jnp.int32, (1, _T), 1)
    m = jnp.maximum(m, jnp.where(col_ids < row_ids, sd, -1.0))
    keep = (jnp.max(m, axis=1, keepdims=True) <= 0.0).astype(jnp.float32)
    out_ref[:, 0:6] = r * keep
    out_ref[:, 6:7] = scores_ref[...] * keep
    out_ref[:, 7:8] = jnp.zeros((_T, 1), jnp.float32)


_nms_call = pl.pallas_call(
    _nms_body,
    grid=(_NSTRIP,),
    in_specs=[
        pl.BlockSpec((_T, 6), lambda i: (i, 0)),
        pl.BlockSpec((_T, 1), lambda i: (i, 0)),
        pl.BlockSpec((6, _KP), lambda i: (0, 0)),
    ],
    out_specs=pl.BlockSpec((_T, 8), lambda i: (i, 0)),
    out_shape=jax.ShapeDtypeStruct((_KP, 8), jnp.float32),
)


@jax.jit
def kernel(boxes, scores):
    top_scores, top_idx = lax.top_k(scores, _KP)
    top_boxes = jnp.take(boxes, top_idx, axis=0)
    out = _nms_call(top_boxes, top_scores[:, None], top_boxes.T)
    return out[:_K, :7]
